# baseline (speedup 1.0000x reference)
import jax
import jax.numpy as jnp
from jax.experimental import pallas as pl


def _head(z_ref, g_ref, b_ref, m_ref, inv_ref, w2_ref, b2_ref, o_ref):
    z = z_ref[...]
    y = jax.nn.relu((z - m_ref[...]) * inv_ref[...] * g_ref[...] + b_ref[...])
    o_ref[...] = jnp.tanh(jnp.dot(y, w2_ref[...], preferred_element_type=jnp.float32) + b2_ref[...])


def kernel(x, edge_index, W1, b1, gamma1, beta1, W2, b2, gamma2, beta2, W3, b3, gamma3, beta3, fc1_W, fc1_b, gamma_fc, beta_fc, fc2_W, fc2_b):
    n = x.shape[0]
    sl = jnp.arange(n, dtype=edge_index.dtype)
    src = jnp.concatenate([edge_index[0], sl])
    dst = jnp.concatenate([edge_index[1], sl])
    deg = jax.ops.segment_sum(jnp.ones_like(src, dtype=jnp.float32), dst, num_segments=n)
    dinv = jnp.where(deg > 0, deg ** -0.5, 0.0)
    norm = dinv[src] * dinv[dst]

    def conv(h, W, b):
        hw = h @ W
        return jax.ops.segment_sum(hw[src] * norm[:, None], dst, num_segments=n) + b

    def bn(h, g, bt):
        m = h.mean(axis=0)
        v = h.var(axis=0)
        return (h - m) / jnp.sqrt(v + 1e-5) * g + bt

    h = jax.nn.relu(bn(conv(x, W1, b1), gamma1, beta1))
    h = jax.nn.relu(bn(conv(h, W2, b2), gamma2, beta2))
    h = jax.nn.relu(bn(conv(h, W3, b3), gamma3, beta3))
    z = h @ fc1_W + fc1_b
    m = z.mean(axis=0)
    inv = jax.lax.rsqrt(z.var(axis=0) + 1e-5)
    NP, R = 50176, 1024
    zp = jnp.zeros((NP, 32), jnp.float32).at[:n].set(z)
    out = pl.pallas_call(
        _head,
        grid=(NP // R,),
        in_specs=[
            pl.BlockSpec((R, 32), lambda i: (i, 0)),
            pl.BlockSpec((1, 32), lambda i: (0, 0)),
            pl.BlockSpec((1, 32), lambda i: (0, 0)),
            pl.BlockSpec((1, 32), lambda i: (0, 0)),
            pl.BlockSpec((1, 32), lambda i: (0, 0)),
            pl.BlockSpec((32, 2), lambda i: (0, 0)),
            pl.BlockSpec((1, 2), lambda i: (0, 0)),
        ],
        out_specs=pl.BlockSpec((R, 2), lambda i: (i, 0)),
        out_shape=jax.ShapeDtypeStruct((NP, 2), jnp.float32),
    )(zp, gamma_fc.reshape(1, 32), beta_fc.reshape(1, 32), m.reshape(1, 32),
      inv.reshape(1, 32), fc2_W, fc2_b.reshape(1, 2))
    return out[:n]


# speedup vs baseline: 6.6607x; 6.6607x over previous
import functools

import jax
import jax.numpy as jnp
from jax import lax
from jax.experimental import pallas as pl
from jax.experimental.pallas import tpu as pltpu
from jax.experimental.pallas import tpu_sc as plsc

NN = 50000
NP = 50176
R = 1024
G = NP // R
EREAL = 800000
EP = 802816
B = 128
NC, NS = 2, 16
ACC_R = 51200
STRIPE = ACC_R // NS
H = 128

@functools.lru_cache(maxsize=None)
def _make_sc_spmm(wacc, c_chunks, split, gather, tab_w):
    mesh = plsc.VectorSubcoreMesh(core_axis_name="c", subcore_axis_name="s",
                                  num_cores=NC, num_subcores=NS)
    ept = EP // (NC * NS) if split else EP // NS
    nsteps = ept // B
    cpc = 1 if split else c_chunks // NC
    n_out = NC if split else c_chunks

    scratch = [
        pltpu.VMEM((B,), jnp.int32),
        pltpu.VMEM((B,), jnp.int32),
        pltpu.VMEM((B, wacc), jnp.float32),
        pltpu.VMEM_SHARED((ACC_R, wacc), jnp.float32),
        pltpu.SemaphoreType.DMA,
    ]

    def body(*refs):
        if gather:
            (table_h, src_h, dst_h, zeros_h, out_h,
             dstbuf, srcbuf, rowbuf, acc, sem) = refs
        else:
            (ones_h, dst_h, zeros_h, out_h,
             dstbuf, srcbuf, rowbuf, acc, sem) = refs
        ci = lax.axis_index("c")
        si = lax.axis_index("s")
        if split:
            ebase = (ci * NS + si) * ept
        else:
            ebase = si * ept
        if not gather:
            pltpu.sync_copy(ones_h, rowbuf)
        for q in range(cpc):
            chunk = ci + NC * q if not split else None
            outidx = ci if split else chunk
            pltpu.sync_copy(zeros_h, acc.at[pl.ds(si * STRIPE, STRIPE)])
            plsc.subcore_barrier()

            if not split:
                chunk_off = jnp.broadcast_to(chunk * NP, (16,)).astype(jnp.int32)

            def step(j, carry):
                eb = ebase + j * B
                pltpu.sync_copy(dst_h.at[pl.ds(eb, B)], dstbuf)
                if gather:
                    pltpu.sync_copy(src_h.at[pl.ds(eb, B)], srcbuf)
                    if tab_w != wacc:
                        for r_ in range(B // 16):
                            sl = pl.ds(r_ * 16, 16)
                            srcbuf[sl] = srcbuf[sl] + chunk_off
                    pltpu.async_copy(table_h.at[srcbuf], rowbuf, sem).wait()
                pltpu.sync_copy(rowbuf, acc.at[dstbuf], add=True)
                return carry

            lax.fori_loop(0, nsteps, step, 0)
            plsc.subcore_barrier()
            pltpu.sync_copy(
                acc.at[pl.ds(si * STRIPE, STRIPE)],
                out_h.at[outidx, pl.ds(si * STRIPE, STRIPE)])

    return pl.kernel(
        body,
        out_type=jax.ShapeDtypeStruct((n_out, ACC_R, wacc), jnp.float32),
        mesh=mesh,
        scratch_types=scratch,
        compiler_params=pltpu.CompilerParams(use_tc_tiling_on_sc=False),
    )


def _blk(w):
    return pl.BlockSpec((R, w), lambda i: (i, 0))


def _rep(w):
    return pl.BlockSpec((1, w), lambda i: (0, 0))


def _full(a, b):
    return pl.BlockSpec((a, b), lambda i: (0, 0))


def _cblk(k, w):
    return pl.BlockSpec((k, R, w), lambda i: (0, i, 0))


def _prep_body(dacc_ref, x_ref, dinv_ref, tbl_ref):
    d = dacc_ref[0][:, 0:1] + dacc_ref[1][:, 0:1] + 1.0
    di = lax.rsqrt(d)
    dinv_ref[...] = di
    tbl_ref[...] = jnp.concatenate(
        [di * x_ref[...], jnp.zeros((R, 30), jnp.float32)], axis=1)


_prep = pl.pallas_call(
    _prep_body,
    grid=(G,),
    in_specs=[_cblk(2, 16), _blk(2)],
    out_specs=[_blk(1), _blk(32)],
    out_shape=[jax.ShapeDtypeStruct((NP, 1), jnp.float32),
               jax.ShapeDtypeStruct((NP, 32), jnp.float32)],
)


def _stats_accum(i, z, sum_ref, sq_ref):
    rid = lax.broadcasted_iota(jnp.int32, (R, 1), 0) + i * R
    zm = jnp.where(rid < NN, z, 0.0)
    ps = jnp.sum(zm, axis=0, keepdims=True)
    psq = jnp.sum(zm * zm, axis=0, keepdims=True)

    @pl.when(i == 0)
    def _():
        sum_ref[...] = ps
        sq_ref[...] = psq

    @pl.when(i > 0)
    def _():
        sum_ref[...] += ps
        sq_ref[...] += psq


def _make_dense(n_agg, combine, chunked_tab, inw, outw):
    def body(agg_ref, tbl_ref, dinv_ref, w_ref, b_ref, z_ref, sum_ref, sq_ref):
        if combine == "sum":
            a = agg_ref[0] + agg_ref[1]
        else:
            a = jnp.concatenate([agg_ref[c] for c in range(n_agg)], axis=-1)
        if chunked_tab:
            t = jnp.concatenate([tbl_ref[c] for c in range(n_agg)], axis=-1)
        else:
            t = tbl_ref[...]
        s = (a + t) * dinv_ref[...]
        z = jnp.dot(s, w_ref[...], preferred_element_type=jnp.float32,
                    precision=lax.Precision.HIGHEST)
        z = z + b_ref[...]
        z_ref[...] = z
        _stats_accum(pl.program_id(0), z, sum_ref, sq_ref)

    tab_spec = _cblk(n_agg, 32) if chunked_tab else _blk(inw)
    return pl.pallas_call(
        body,
        grid=(G,),
        in_specs=[_cblk(n_agg, 32), tab_spec, _blk(1), _full(inw, outw), _rep(outw)],
        out_specs=[_blk(outw), _rep(outw), _rep(outw)],
        out_shape=[jax.ShapeDtypeStruct((NP, outw), jnp.float32),
                   jax.ShapeDtypeStruct((1, outw), jnp.float32),
                   jax.ShapeDtypeStruct((1, outw), jnp.float32)],
    )


def _bn_core(z_ref, sum_ref, sq_ref, g_ref, bt_ref):
    m = sum_ref[...] * (1.0 / NN)
    v = sq_ref[...] * (1.0 / NN) - m * m
    inv = lax.rsqrt(v + 1e-5)
    return jnp.maximum((z_ref[...] - m) * inv * g_ref[...] + bt_ref[...], 0.0)


def _bnapply_body(z_ref, sum_ref, sq_ref, g_ref, bt_ref, dinv_ref, out_ref):
    ys = _bn_core(z_ref, sum_ref, sq_ref, g_ref, bt_ref) * dinv_ref[...]
    for c in range(4):
        out_ref[c] = ys[:, 32 * c:32 * c + 32]


_bnapply = pl.pallas_call(
    _bnapply_body,
    grid=(G,),
    in_specs=[_blk(H), _rep(H), _rep(H), _rep(H), _rep(H), _blk(1)],
    out_specs=_cblk(4, 32),
    out_shape=jax.ShapeDtypeStruct((4, NP, 32), jnp.float32),
)


def _bnmm_body(z_ref, sum_ref, sq_ref, g_ref, bt_ref, w_ref, b_ref,
               z1_ref, sum1_ref, sq1_ref):
    y = _bn_core(z_ref, sum_ref, sq_ref, g_ref, bt_ref)
    z1 = jnp.dot(y, w_ref[...], preferred_element_type=jnp.float32,
                 precision=lax.Precision.HIGHEST) + b_ref[...]
    z1_ref[...] = z1
    _stats_accum(pl.program_id(0), z1, sum1_ref, sq1_ref)


_bnmm = pl.pallas_call(
    _bnmm_body,
    grid=(G,),
    in_specs=[_blk(H), _rep(H), _rep(H), _rep(H), _rep(H), _full(H, 32), _rep(32)],
    out_specs=[_blk(32), _rep(32), _rep(32)],
    out_shape=[jax.ShapeDtypeStruct((NP, 32), jnp.float32),
               jax.ShapeDtypeStruct((1, 32), jnp.float32),
               jax.ShapeDtypeStruct((1, 32), jnp.float32)],
)


def _final_body(z_ref, sum_ref, sq_ref, g_ref, bt_ref, w_ref, b_ref, o_ref):
    y = _bn_core(z_ref, sum_ref, sq_ref, g_ref, bt_ref)
    o_ref[...] = jnp.tanh(
        jnp.dot(y, w_ref[...], preferred_element_type=jnp.float32,
                precision=lax.Precision.HIGHEST) + b_ref[...])


_final = pl.pallas_call(
    _final_body,
    grid=(G,),
    in_specs=[_blk(32), _rep(32), _rep(32), _rep(32), _rep(32), _full(32, 2), _rep(2)],
    out_specs=_blk(2),
    out_shape=jax.ShapeDtypeStruct((NP, 2), jnp.float32),
)

_dense1 = _make_dense(n_agg=2, combine="sum", chunked_tab=False, inw=32, outw=H)
_dense2 = _make_dense(n_agg=4, combine="concat", chunked_tab=True, inw=H, outw=H)


def kernel(x, edge_index, W1, b1, gamma1, beta1, W2, b2, gamma2, beta2,
           W3, b3, gamma3, beta3, fc1_W, fc1_b, gamma_fc, beta_fc, fc2_W, fc2_b):
    f32 = jnp.float32
    _deg_sc = _make_sc_spmm(wacc=16, c_chunks=1, split=True, gather=False, tab_w=16)
    _spmm1_sc = _make_sc_spmm(wacc=32, c_chunks=1, split=True, gather=True, tab_w=32)
    _spmm_sc = _make_sc_spmm(wacc=32, c_chunks=4, split=False, gather=True, tab_w=H)
    pad = jnp.full((EP - EREAL,), NN, jnp.int32)
    src = jnp.concatenate([edge_index[0].astype(jnp.int32), pad])
    dst = jnp.concatenate([edge_index[1].astype(jnp.int32), pad])
    x_pad = jnp.zeros((NP, 2), f32).at[:NN].set(x)
    ones16 = jnp.ones((B, 16), f32)
    z16 = jnp.zeros((STRIPE, 16), f32)
    z32 = jnp.zeros((STRIPE, 32), f32)

    degacc = _deg_sc(ones16, dst, z16)
    dinv, tbl1 = _prep(degacc, x_pad)
    agg1 = _spmm1_sc(tbl1, src, dst, z32)
    W1p = jnp.zeros((32, H), f32).at[:2].set(W1)
    z1, s1, q1 = _dense1(agg1, tbl1, dinv, W1p, b1.reshape(1, H))
    tbl2 = _bnapply(z1, s1, q1, gamma1.reshape(1, H), beta1.reshape(1, H), dinv)
    agg2 = _spmm_sc(tbl2.reshape(4 * NP, 32), src, dst, z32)
    z2, s2, q2 = _dense2(agg2, tbl2, dinv, W2, b2.reshape(1, H))
    tbl3 = _bnapply(z2, s2, q2, gamma2.reshape(1, H), beta2.reshape(1, H), dinv)
    agg3 = _spmm_sc(tbl3.reshape(4 * NP, 32), src, dst, z32)
    z3, s3, q3 = _dense2(agg3, tbl3, dinv, W3, b3.reshape(1, H))
    zf, sf, qf = _bnmm(z3, s3, q3, gamma3.reshape(1, H), beta3.reshape(1, H),
                       fc1_W, fc1_b.reshape(1, 32))
    out = _final(zf, sf, qf, gamma_fc.reshape(1, 32), beta_fc.reshape(1, 32),
                 fc2_W, fc2_b.reshape(1, 2))
    return out[:NN]


# speedup vs baseline: 12.0055x; 1.8024x over previous
import functools

import jax
import jax.numpy as jnp
from jax import lax
from jax.experimental import pallas as pl
from jax.experimental.pallas import tpu as pltpu
from jax.experimental.pallas import tpu_sc as plsc

NN = 50000
NP = 50176
R = 1024
G = NP // R
EREAL = 800000
EP = 802816
B = 128
NC, NS = 2, 16
ACC_R = 50176
STRIPE = ACC_R // NS
H = 128

@functools.lru_cache(maxsize=None)
def _make_sc_spmm(wacc, c_chunks, split, gather, tab_w):
    mesh = plsc.VectorSubcoreMesh(core_axis_name="c", subcore_axis_name="s",
                                  num_cores=NC, num_subcores=NS)
    ept = EP // (NC * NS) if split else EP // NS
    nsteps = ept // B
    cpc = 1 if split else c_chunks // NC
    n_out = NC if split else c_chunks
    K = 4
    ngroups = nsteps // K

    scratch = [[pltpu.VMEM((2, B), jnp.int32) for _ in range(K)],
               [pltpu.VMEM((B, wacc), jnp.float32) for _ in range(K)],
               pltpu.VMEM_SHARED((ACC_R, wacc), jnp.float32),
               pltpu.SemaphoreType.DMA]

    def body(*refs):
        if gather:
            table_h, idx_h, zeros_h, out_h, idxbufs, rowbufs, acc, sem = refs
        else:
            ones_h, idx_h, zeros_h, out_h, idxbufs, rowbufs, acc, sem = refs
        ci = lax.axis_index("c")
        si = lax.axis_index("s")
        bb = ((ci * NS + si) if split else si) * nsteps
        if not gather:
            for b in range(K):
                pltpu.sync_copy(ones_h, rowbufs[b])
        for q in range(cpc):
            chunk = ci + NC * q if not split else None
            outidx = ci if split else chunk
            pltpu.sync_copy(zeros_h, acc.at[pl.ds(si * STRIPE, STRIPE)])
            plsc.subcore_barrier()

            if gather and tab_w != wacc:
                chunk_off = jnp.broadcast_to(chunk * NP, (16,)).astype(jnp.int32)

            def fire(g, b):
                idxb, rowb = idxbufs[b], rowbufs[b]
                pltpu.sync_copy(idx_h.at[bb + g * K + b], idxb)
                if gather:
                    if tab_w != wacc:
                        for r_ in range(B // 16):
                            sl = pl.ds(r_ * 16, 16)
                            idxb[0, sl] = idxb[0, sl] + chunk_off
                    pltpu.async_copy(table_h.at[idxb.at[0]], rowb, sem)

            for b in range(K):
                fire(0, b)

            def grp(g, carry):
                for b in range(K):
                    idxb, rowb = idxbufs[b], rowbufs[b]
                    if gather:
                        pltpu.make_async_copy(
                            table_h.at[idxb.at[0]], rowb, sem).wait()
                    pltpu.sync_copy(rowb, acc.at[idxb.at[1]], add=True)

                    @pl.when(g + 1 < ngroups)
                    def _():
                        fire(g + 1, b)
                return carry

            lax.fori_loop(0, ngroups, grp, 0)
            plsc.subcore_barrier()
            pltpu.sync_copy(
                acc.at[pl.ds(si * STRIPE, STRIPE)],
                out_h.at[outidx, pl.ds(si * STRIPE, STRIPE)])

    return pl.kernel(
        body,
        out_type=jax.ShapeDtypeStruct((n_out, ACC_R, wacc), jnp.float32),
        mesh=mesh,
        scratch_types=scratch,
        compiler_params=pltpu.CompilerParams(use_tc_tiling_on_sc=False),
    )


def _blk(w):
    return pl.BlockSpec((R, w), lambda i: (i, 0))


def _rep(w):
    return pl.BlockSpec((1, w), lambda i: (0, 0))


def _full(a, b):
    return pl.BlockSpec((a, b), lambda i: (0, 0))


def _cblk(k, w):
    return pl.BlockSpec((k, R, w), lambda i: (0, i, 0))


def _prep_body(dacc_ref, x_ref, dinv_ref, tbl_ref):
    d = dacc_ref[0][:, 0:1] + dacc_ref[1][:, 0:1] + 1.0
    di = lax.rsqrt(d)
    dinv_ref[...] = di
    tbl_ref[...] = jnp.concatenate(
        [di * x_ref[...], jnp.zeros((R, 30), jnp.float32)], axis=1)


_prep = pl.pallas_call(
    _prep_body,
    grid=(G,),
    in_specs=[_cblk(2, 16), _blk(2)],
    out_specs=[_blk(1), _blk(32)],
    out_shape=[jax.ShapeDtypeStruct((NP, 1), jnp.float32),
               jax.ShapeDtypeStruct((NP, 32), jnp.float32)],
)


def _stats_accum(i, z, sum_ref, sq_ref):
    rid = lax.broadcasted_iota(jnp.int32, (R, 1), 0) + i * R
    zm = jnp.where(rid < NN, z, 0.0)
    ps = jnp.sum(zm, axis=0, keepdims=True)
    psq = jnp.sum(zm * zm, axis=0, keepdims=True)

    @pl.when(i == 0)
    def _():
        sum_ref[...] = ps
        sq_ref[...] = psq

    @pl.when(i > 0)
    def _():
        sum_ref[...] += ps
        sq_ref[...] += psq


def _make_dense(n_agg, combine, chunked_tab, inw, outw):
    def body(agg_ref, tbl_ref, dinv_ref, w_ref, b_ref, z_ref, sum_ref, sq_ref):
        if combine == "sum":
            a = agg_ref[0] + agg_ref[1]
        else:
            a = jnp.concatenate([agg_ref[c] for c in range(n_agg)], axis=-1)
        if chunked_tab:
            t = jnp.concatenate([tbl_ref[c] for c in range(n_agg)], axis=-1)
        else:
            t = tbl_ref[...]
        s = (a + t) * dinv_ref[...]
        z = jnp.dot(s, w_ref[...], preferred_element_type=jnp.float32,
                    precision=lax.Precision.HIGHEST)
        z = z + b_ref[...]
        z_ref[...] = z
        _stats_accum(pl.program_id(0), z, sum_ref, sq_ref)

    tab_spec = _cblk(n_agg, 32) if chunked_tab else _blk(inw)
    return pl.pallas_call(
        body,
        grid=(G,),
        in_specs=[_cblk(n_agg, 32), tab_spec, _blk(1), _full(inw, outw), _rep(outw)],
        out_specs=[_blk(outw), _rep(outw), _rep(outw)],
        out_shape=[jax.ShapeDtypeStruct((NP, outw), jnp.float32),
                   jax.ShapeDtypeStruct((1, outw), jnp.float32),
                   jax.ShapeDtypeStruct((1, outw), jnp.float32)],
    )


def _bn_core(z_ref, sum_ref, sq_ref, g_ref, bt_ref):
    m = sum_ref[...] * (1.0 / NN)
    v = sq_ref[...] * (1.0 / NN) - m * m
    inv = lax.rsqrt(v + 1e-5)
    return jnp.maximum((z_ref[...] - m) * inv * g_ref[...] + bt_ref[...], 0.0)


def _bnapply_body(z_ref, sum_ref, sq_ref, g_ref, bt_ref, dinv_ref, out_ref):
    ys = _bn_core(z_ref, sum_ref, sq_ref, g_ref, bt_ref) * dinv_ref[...]
    for c in range(4):
        out_ref[c] = ys[:, 32 * c:32 * c + 32]


_bnapply = pl.pallas_call(
    _bnapply_body,
    grid=(G,),
    in_specs=[_blk(H), _rep(H), _rep(H), _rep(H), _rep(H), _blk(1)],
    out_specs=_cblk(4, 32),
    out_shape=jax.ShapeDtypeStruct((4, NP, 32), jnp.float32),
)


def _bnmm_body(z_ref, sum_ref, sq_ref, g_ref, bt_ref, w_ref, b_ref,
               z1_ref, sum1_ref, sq1_ref):
    y = _bn_core(z_ref, sum_ref, sq_ref, g_ref, bt_ref)
    z1 = jnp.dot(y, w_ref[...], preferred_element_type=jnp.float32,
                 precision=lax.Precision.HIGHEST) + b_ref[...]
    z1_ref[...] = z1
    _stats_accum(pl.program_id(0), z1, sum1_ref, sq1_ref)


_bnmm = pl.pallas_call(
    _bnmm_body,
    grid=(G,),
    in_specs=[_blk(H), _rep(H), _rep(H), _rep(H), _rep(H), _full(H, 32), _rep(32)],
    out_specs=[_blk(32), _rep(32), _rep(32)],
    out_shape=[jax.ShapeDtypeStruct((NP, 32), jnp.float32),
               jax.ShapeDtypeStruct((1, 32), jnp.float32),
               jax.ShapeDtypeStruct((1, 32), jnp.float32)],
)


def _final_body(z_ref, sum_ref, sq_ref, g_ref, bt_ref, w_ref, b_ref, o_ref):
    y = _bn_core(z_ref, sum_ref, sq_ref, g_ref, bt_ref)
    o_ref[...] = jnp.tanh(
        jnp.dot(y, w_ref[...], preferred_element_type=jnp.float32,
                precision=lax.Precision.HIGHEST) + b_ref[...])


_final = pl.pallas_call(
    _final_body,
    grid=(G,),
    in_specs=[_blk(32), _rep(32), _rep(32), _rep(32), _rep(32), _full(32, 2), _rep(2)],
    out_specs=_blk(2),
    out_shape=jax.ShapeDtypeStruct((NP, 2), jnp.float32),
)

_dense1 = _make_dense(n_agg=2, combine="sum", chunked_tab=False, inw=32, outw=H)
_dense2 = _make_dense(n_agg=4, combine="concat", chunked_tab=True, inw=H, outw=H)


def kernel(x, edge_index, W1, b1, gamma1, beta1, W2, b2, gamma2, beta2,
           W3, b3, gamma3, beta3, fc1_W, fc1_b, gamma_fc, beta_fc, fc2_W, fc2_b):
    f32 = jnp.float32
    _deg_sc = _make_sc_spmm(wacc=16, c_chunks=1, split=True, gather=False, tab_w=16)
    _spmm1_sc = _make_sc_spmm(wacc=32, c_chunks=1, split=True, gather=True, tab_w=32)
    _spmm_sc = _make_sc_spmm(wacc=32, c_chunks=4, split=False, gather=True, tab_w=H)
    pad = jnp.full((EP - EREAL,), NN, jnp.int32)
    src = jnp.concatenate([edge_index[0].astype(jnp.int32), pad])
    dst = jnp.concatenate([edge_index[1].astype(jnp.int32), pad])
    idx3 = jnp.stack([src.reshape(-1, B), dst.reshape(-1, B)], axis=1)
    x_pad = jnp.zeros((NP, 2), f32).at[:NN].set(x)
    ones16 = jnp.ones((B, 16), f32)
    z16 = jnp.zeros((STRIPE, 16), f32)
    z32 = jnp.zeros((STRIPE, 32), f32)

    degacc = _deg_sc(ones16, idx3, z16)
    dinv, tbl1 = _prep(degacc, x_pad)
    agg1 = _spmm1_sc(tbl1, idx3, z32)
    W1p = jnp.zeros((32, H), f32).at[:2].set(W1)
    z1, s1, q1 = _dense1(agg1, tbl1, dinv, W1p, b1.reshape(1, H))
    tbl2 = _bnapply(z1, s1, q1, gamma1.reshape(1, H), beta1.reshape(1, H), dinv)
    agg2 = _spmm_sc(tbl2.reshape(4 * NP, 32), idx3, z32)
    z2, s2, q2 = _dense2(agg2, tbl2, dinv, W2, b2.reshape(1, H))
    tbl3 = _bnapply(z2, s2, q2, gamma2.reshape(1, H), beta2.reshape(1, H), dinv)
    agg3 = _spmm_sc(tbl3.reshape(4 * NP, 32), idx3, z32)
    z3, s3, q3 = _dense2(agg3, tbl3, dinv, W3, b3.reshape(1, H))
    zf, sf, qf = _bnmm(z3, s3, q3, gamma3.reshape(1, H), beta3.reshape(1, H),
                       fc1_W, fc1_b.reshape(1, 32))
    out = _final(zf, sf, qf, gamma_fc.reshape(1, 32), beta_fc.reshape(1, 32),
                 fc2_W, fc2_b.reshape(1, 2))
    return out[:NN]
